# baseline (speedup 1.0000x reference)
import functools

import jax
import jax.numpy as jnp
from jax import lax
from jax.experimental import pallas as pl
from jax.experimental.pallas import tpu as pltpu
from jax.experimental.pallas import tpu_sc as plsc

_BINS = (-1.1503, -0.6745, -0.3186, 0.0, 0.3186, 0.6745, 1.1503)

_N = 32768
_LANES = 16
_SUB = 8
_LD = 1024
_ROUND = 32
_RELEM = _ROUND * _SUB


def _bucketize(v):
    acc = (v > _BINS[0]).astype(jnp.int32)
    for b in _BINS[1:]:
        acc = acc + (v > b).astype(jnp.int32)
    return acc


def _sc_kernel(bpw: int, x_hbm, out_hbm, buf0, buf1, res_v,
               sem0, sem1):
    nc = plsc.get_sparse_core_info().num_cores
    wid = lax.axis_index("s") * nc + lax.axis_index("c")
    base = pl.multiple_of(wid * bpw, bpw)
    blk0 = pl.multiple_of(wid * (bpw // _SUB), bpw // _SUB)

    lanes = lax.iota(jnp.int32, _LANES)
    zeros = lanes * 0
    n_blocks = bpw // _SUB

    bufs = (buf0, buf1)
    sems = (sem0, sem1)

    def fire(rnd):
        return pltpu.async_copy(
            x_hbm.at[pl.ds(blk0 + rnd * _ROUND, _ROUND), :, pl.ds(0, 128)],
            bufs[rnd % 2], sems[rnd % 2])

    n_rounds = n_blocks // _ROUND
    cps = [fire(0), None]
    for rnd in range(n_rounds):
        if rnd + 1 < n_rounds:
            cps[(rnd + 1) % 2] = fire(rnd + 1)
        cps[rnd % 2].wait()
        buf = bufs[rnd % 2]
        for g in range(_RELEM // _LANES):
            e = g * _LANES + lanes
            v = plsc.load_gather(
                buf, [lax.shift_right_logical(e, 3),
                      lax.bitwise_and(e, _SUB - 1), zeros])
            res_v[pl.ds(rnd * _RELEM + g * _LANES, _LANES)] = _bucketize(v)

    pltpu.sync_copy(res_v, out_hbm.at[pl.ds(base, bpw)])


def kernel(x):
    info = plsc.get_sparse_core_info()
    n_workers = info.num_cores * info.num_subcores
    bpw = _N // n_workers

    x3 = x.reshape(_N * 2 // _SUB, _SUB, _LD)

    mesh = plsc.VectorSubcoreMesh(core_axis_name="c", subcore_axis_name="s")
    k = functools.partial(
        pl.kernel,
        mesh=mesh,
        compiler_params=pltpu.CompilerParams(needs_layout_passes=False, skip_device_barrier=True, disable_bounds_checks=True, disable_semaphore_checks=True),
        out_type=jax.ShapeDtypeStruct((_N,), jnp.int32),
        scratch_types=[
            pltpu.VMEM((_ROUND, _SUB, 128), jnp.float32),
            pltpu.VMEM((_ROUND, _SUB, 128), jnp.float32),
            pltpu.VMEM((bpw,), jnp.int32),
            pltpu.SemaphoreType.DMA,
            pltpu.SemaphoreType.DMA,
        ],
    )(functools.partial(_sc_kernel, bpw))

    return k(x3)


# speedup vs baseline: 1.0237x; 1.0237x over previous
import functools

import jax
import jax.numpy as jnp
from jax import lax
from jax.experimental import pallas as pl
from jax.experimental.pallas import tpu as pltpu
from jax.experimental.pallas import tpu_sc as plsc

_BINS = (-1.1503, -0.6745, -0.3186, 0.0, 0.3186, 0.6745, 1.1503)

_N = 32768
_LANES = 16
_SUB = 8
_LD = 1024
_ROUND = 32
_RELEM = _ROUND * _SUB
_NBUF = 3


def _bucketize(v):
    acc = (v > _BINS[0]).astype(jnp.int32)
    for b in _BINS[1:]:
        acc = acc + (v > b).astype(jnp.int32)
    return acc


def _sc_kernel(bpw: int, x_hbm, out_hbm, buf0, buf1, buf2, res_v,
               sem0, sem1, sem2, out_sem):
    nc = plsc.get_sparse_core_info().num_cores
    wid = lax.axis_index("s") * nc + lax.axis_index("c")
    base = pl.multiple_of(wid * bpw, bpw)
    blk0 = pl.multiple_of(wid * (bpw // _SUB), bpw // _SUB)

    lanes = lax.iota(jnp.int32, _LANES)
    zeros = lanes * 0
    n_blocks = bpw // _SUB

    bufs = (buf0, buf1, buf2)
    sems = (sem0, sem1, sem2)

    def fire(rnd):
        return pltpu.async_copy(
            x_hbm.at[pl.ds(blk0 + rnd * _ROUND, _ROUND), :, pl.ds(0, 128)],
            bufs[rnd % _NBUF], sems[rnd % _NBUF])

    n_rounds = n_blocks // _ROUND
    depth = min(_NBUF - 1, n_rounds)
    cps = [None] * _NBUF
    for rnd in range(depth):
        cps[rnd % _NBUF] = fire(rnd)
    out_cps = []
    for rnd in range(n_rounds):
        if rnd + depth < n_rounds:
            cps[(rnd + depth) % _NBUF] = fire(rnd + depth)
        cps[rnd % _NBUF].wait()
        buf = bufs[rnd % _NBUF]
        for g in range(_RELEM // _LANES):
            e = g * _LANES + lanes
            v = plsc.load_gather(
                buf, [lax.shift_right_logical(e, 3),
                      lax.bitwise_and(e, _SUB - 1), zeros])
            res_v[pl.ds(rnd * _RELEM + g * _LANES, _LANES)] = _bucketize(v)
        out_cps.append(
            pltpu.async_copy(
                res_v.at[pl.ds(rnd * _RELEM, _RELEM)],
                out_hbm.at[pl.ds(base + rnd * _RELEM, _RELEM)], out_sem))
    for cp in out_cps:
        cp.wait()


def kernel(x):
    info = plsc.get_sparse_core_info()
    n_workers = info.num_cores * info.num_subcores
    bpw = _N // n_workers

    x3 = x.reshape(_N * 2 // _SUB, _SUB, _LD)

    mesh = plsc.VectorSubcoreMesh(core_axis_name="c", subcore_axis_name="s")
    k = functools.partial(
        pl.kernel,
        mesh=mesh,
        compiler_params=pltpu.CompilerParams(needs_layout_passes=False),
        out_type=jax.ShapeDtypeStruct((_N,), jnp.int32),
        scratch_types=[
            pltpu.VMEM((_ROUND, _SUB, 128), jnp.float32),
            pltpu.VMEM((_ROUND, _SUB, 128), jnp.float32),
            pltpu.VMEM((_ROUND, _SUB, 128), jnp.float32),
            pltpu.VMEM((bpw,), jnp.int32),
            pltpu.SemaphoreType.DMA,
            pltpu.SemaphoreType.DMA,
            pltpu.SemaphoreType.DMA,
            pltpu.SemaphoreType.DMA,
        ],
    )(functools.partial(_sc_kernel, bpw))

    return k(x3)


# speedup vs baseline: 1.0425x; 1.0184x over previous
import functools

import jax
import jax.numpy as jnp
from jax import lax
from jax.experimental import pallas as pl
from jax.experimental.pallas import tpu as pltpu
from jax.experimental.pallas import tpu_sc as plsc

_BINS = (-1.1503, -0.6745, -0.3186, 0.0, 0.3186, 0.6745, 1.1503)

_N = 32768
_LANES = 16
_SUB = 8
_LD = 1024
_ROUND = 16
_RELEM = _ROUND * _SUB
_NBUF = 6


def _bucketize(v):
    acc = (v > _BINS[0]).astype(jnp.int32)
    for b in _BINS[1:]:
        acc = acc + (v > b).astype(jnp.int32)
    return acc


def _sc_kernel(bpw: int, x_hbm, out_hbm, buf0, buf1, buf2, buf3, buf4,
               buf5, res_v, sem0, sem1, sem2, sem3, sem4, sem5, out_sem):
    nc = plsc.get_sparse_core_info().num_cores
    wid = lax.axis_index("s") * nc + lax.axis_index("c")
    base = pl.multiple_of(wid * bpw, bpw)
    blk0 = pl.multiple_of(wid * (bpw // _SUB), bpw // _SUB)

    lanes = lax.iota(jnp.int32, _LANES)
    zeros = lanes * 0
    n_blocks = bpw // _SUB

    bufs = (buf0, buf1, buf2, buf3, buf4, buf5)
    sems = (sem0, sem1, sem2, sem3, sem4, sem5)

    def fire(rnd):
        return pltpu.async_copy(
            x_hbm.at[pl.ds(blk0 + rnd * _ROUND, _ROUND), :, pl.ds(0, 128)],
            bufs[rnd % _NBUF], sems[rnd % _NBUF])

    n_rounds = n_blocks // _ROUND
    depth = min(_NBUF - 1, n_rounds)
    cps = [None] * _NBUF
    for rnd in range(depth):
        cps[rnd % _NBUF] = fire(rnd)
    out_cps = []
    for rnd in range(n_rounds):
        if rnd + depth < n_rounds:
            cps[(rnd + depth) % _NBUF] = fire(rnd + depth)
        cps[rnd % _NBUF].wait()
        buf = bufs[rnd % _NBUF]
        for g in range(_RELEM // _LANES):
            e = g * _LANES + lanes
            v = plsc.load_gather(
                buf, [lax.shift_right_logical(e, 3),
                      lax.bitwise_and(e, _SUB - 1), zeros])
            res_v[pl.ds(rnd * _RELEM + g * _LANES, _LANES)] = _bucketize(v)
        out_cps.append(
            pltpu.async_copy(
                res_v.at[pl.ds(rnd * _RELEM, _RELEM)],
                out_hbm.at[pl.ds(base + rnd * _RELEM, _RELEM)], out_sem))
    for cp in out_cps:
        cp.wait()


def kernel(x):
    info = plsc.get_sparse_core_info()
    n_workers = info.num_cores * info.num_subcores
    bpw = _N // n_workers

    x3 = x.reshape(_N * 2 // _SUB, _SUB, _LD)

    mesh = plsc.VectorSubcoreMesh(core_axis_name="c", subcore_axis_name="s")
    k = functools.partial(
        pl.kernel,
        mesh=mesh,
        compiler_params=pltpu.CompilerParams(needs_layout_passes=False),
        out_type=jax.ShapeDtypeStruct((_N,), jnp.int32),
        scratch_types=[
            pltpu.VMEM((_ROUND, _SUB, 128), jnp.float32),
            pltpu.VMEM((_ROUND, _SUB, 128), jnp.float32),
            pltpu.VMEM((_ROUND, _SUB, 128), jnp.float32),
            pltpu.VMEM((_ROUND, _SUB, 128), jnp.float32),
            pltpu.VMEM((_ROUND, _SUB, 128), jnp.float32),
            pltpu.VMEM((_ROUND, _SUB, 128), jnp.float32),
            pltpu.VMEM((bpw,), jnp.int32),
            pltpu.SemaphoreType.DMA,
            pltpu.SemaphoreType.DMA,
            pltpu.SemaphoreType.DMA,
            pltpu.SemaphoreType.DMA,
            pltpu.SemaphoreType.DMA,
            pltpu.SemaphoreType.DMA,
            pltpu.SemaphoreType.DMA,
        ],
    )(functools.partial(_sc_kernel, bpw))

    return k(x3)
